# Initial kernel scaffold; baseline (speedup 1.0000x reference)
import jax
import jax.numpy as jnp
from jax.experimental import pallas as pl


def kernel(inputs, hidden, edges, node_masks, send_edges, recv_edges, edge2node_inds, emb_w, emb_b, edge_w1, edge_b1, edge_w2, edge_b2, node_w1, node_b1, node_w2, node_b2, coord_w1, coord_b1, coord_w2, vel_w1, vel_b1, vel_w2, vel_b2):
    raise NotImplementedError("write your pallas kernel here")


# speedup vs baseline: 4.5139x; 4.5139x over previous
import functools

import jax
import jax.numpy as jnp
from jax import lax
from jax.experimental import pallas as pl
from jax.experimental.pallas import tpu as pltpu
from jax.experimental.pallas import tpu_sc as plsc

F32 = jnp.float32
NC = 2
NS = 16
NW = NC * NS
DW = 16
PW = 128


def _silu(x):
    return x * jax.nn.sigmoid(x)


def _full_spec(shape):
    nd = len(shape)
    return pl.BlockSpec(shape, lambda i, _nd=nd: (0,) * _nd)


def _init_body(xp, xv, embw, embb, w1a, w1b, h_o, t_o, u_o):
    v = xv[...]
    p = xp[...]
    nrm = jnp.sqrt(jnp.sum(v * v, axis=-1, keepdims=True))
    h = nrm * embw[...] + embb[...]
    h_o[...] = h
    z = jnp.zeros((p.shape[0], PW - h.shape[1] - DW), F32)
    ga = jnp.dot(h, w1a[...], preferred_element_type=F32)
    gb = jnp.dot(h, w1b[...], preferred_element_type=F32)
    t_o[...] = jnp.concatenate([ga, p, z], axis=-1)
    u_o[...] = jnp.concatenate([gb, -p, z], axis=-1)


def _tc_init(xp, xv, embw, embb, w1a, w1b, N, H, BN):
    grid = N // BN
    bs_n16 = pl.BlockSpec((BN, DW), lambda i: (i, 0))
    bs_nh = pl.BlockSpec((BN, H), lambda i: (i, 0))
    bs_np = pl.BlockSpec((BN, PW), lambda i: (i, 0))
    return pl.pallas_call(
        _init_body,
        grid=(grid,),
        in_specs=[bs_n16, bs_n16, _full_spec(embw.shape), _full_spec(embb.shape),
                  _full_spec(w1a.shape), _full_spec(w1b.shape)],
        out_specs=[bs_nh, bs_np, bs_np],
        out_shape=[jax.ShapeDtypeStruct((N, H), F32),
                   jax.ShapeDtypeStruct((N, PW), F32),
                   jax.ShapeDtypeStruct((N, PW), F32)],
    )(xp, xv, embw, embb, w1a, w1b)


def _edge_body(H, y, w1c, b1, w2, b2, cw1, cb1, cw2r, e2_o):
    Y = y[...]
    X = Y[:, :H]
    D = Y[:, H:H + DW]
    lane = lax.broadcasted_iota(jnp.int32, D.shape, 1)
    radial = jnp.sum(jnp.where(lane < 3, D * D, 0.0), axis=-1, keepdims=True)
    x1 = X + radial * w1c[...] + b1[...]
    e1 = _silu(x1)
    ef = _silu(jnp.dot(e1, w2[...], preferred_element_type=F32) + b2[...])
    c = _silu(jnp.dot(ef, cw1[...], preferred_element_type=F32) + cb1[...])
    t = jnp.sum(c * cw2r[...], axis=-1, keepdims=True)
    trans = jnp.clip(D * t, -100.0, 100.0)
    s = jnp.where(lane == 3, 1.0, trans)
    z = jnp.zeros((Y.shape[0], PW - H - DW), F32)
    e2_o[...] = jnp.concatenate([ef, s, z], axis=-1)


def _tc_edge(y, w1c, b1, w2, b2, cw1, cb1, cw2r, E, H, BE):
    grid = E // BE
    bs_ep = pl.BlockSpec((BE, PW), lambda i: (i, 0))
    return pl.pallas_call(
        functools.partial(_edge_body, H),
        grid=(grid,),
        in_specs=[bs_ep] + [_full_spec(w.shape)
                            for w in (w1c, b1, w2, b2, cw1, cb1, cw2r)],
        out_specs=[bs_ep],
        out_shape=[jax.ShapeDtypeStruct((E, PW), F32)],
    )(y, w1c, b1, w2, b2, cw1, cb1, cw2r)[0]


def _node_body(emit_next, H, *refs):
    if emit_next:
        (h, pos, vel, p2, vw1, vb1, vw2r, vb2, nw1a, nw1b, nb1, nw2, nb2,
         xa, xb, h_o, pos_o, vel_o, t_o, u_o) = refs
    else:
        (h, pos, vel, p2, vw1, vb1, vw2r, vb2, nw1a, nw1b, nb1, nw2, nb2,
         h_o, pos_o, vel_o) = refs
    hh = h[...]
    P = p2[0] + p2[1]
    agg_e = P[:, :H]
    ag = P[:, H:H + DW]
    lane = lax.broadcasted_iota(jnp.int32, ag.shape, 1)
    cnt = ag[:, 3:4]
    new_vel = jnp.where(lane < 3, ag, 0.0) / jnp.maximum(cnt, 1.0)
    hv = _silu(jnp.dot(hh, vw1[...], preferred_element_type=F32) + vb1[...])
    mv = jnp.sum(hv * vw2r[...], axis=-1, keepdims=True) + vb2[...]
    new_vel = new_vel + mv * vel[...]
    pos2 = pos[...] + new_vel
    pos_o[...] = pos2
    vel_o[...] = new_vel
    pre = _silu(jnp.dot(hh, nw1a[...], preferred_element_type=F32)
                + jnp.dot(agg_e, nw1b[...], preferred_element_type=F32)
                + nb1[...])
    h2 = jnp.dot(pre, nw2[...], preferred_element_type=F32) + nb2[...] + hh
    h_o[...] = h2
    if emit_next:
        z = jnp.zeros((hh.shape[0], PW - H - DW), F32)
        ga = jnp.dot(h2, xa[...], preferred_element_type=F32)
        gb = jnp.dot(h2, xb[...], preferred_element_type=F32)
        t_o[...] = jnp.concatenate([ga, pos2, z], axis=-1)
        u_o[...] = jnp.concatenate([gb, -pos2, z], axis=-1)


def _tc_node(h, pos, vel, p2, vw1, vb1, vw2r, vb2,
             nw1a, nw1b, nb1, nw2, nb2, nxt, N, H, BN):
    grid = N // BN
    bs_nh = pl.BlockSpec((BN, H), lambda i: (i, 0))
    bs_n16 = pl.BlockSpec((BN, DW), lambda i: (i, 0))
    bs_np = pl.BlockSpec((BN, PW), lambda i: (i, 0))
    bs_p2 = pl.BlockSpec((NC, BN, PW), lambda i: (0, i, 0))
    ws = (vw1, vb1, vw2r, vb2, nw1a, nw1b, nb1, nw2, nb2)
    emit_next = nxt is not None
    ins = [h, pos, vel, p2, *ws]
    in_specs = [bs_nh, bs_n16, bs_n16, bs_p2] + [_full_spec(w.shape)
                                                 for w in ws]
    out_specs = [bs_nh, bs_n16, bs_n16]
    out_shape = [jax.ShapeDtypeStruct((N, H), F32),
                 jax.ShapeDtypeStruct((N, DW), F32),
                 jax.ShapeDtypeStruct((N, DW), F32)]
    if emit_next:
        ins += [nxt[0], nxt[1]]
        in_specs += [_full_spec(nxt[0].shape), _full_spec(nxt[1].shape)]
        out_specs += [bs_np, bs_np]
        out_shape += [jax.ShapeDtypeStruct((N, PW), F32),
                      jax.ShapeDtypeStruct((N, PW), F32)]
    return pl.pallas_call(
        functools.partial(_node_body, emit_next, H),
        grid=(grid,),
        in_specs=in_specs,
        out_specs=out_specs,
        out_shape=out_shape,
    )(*ins)


@functools.lru_cache(maxsize=None)
def _build_sc_gather(E, N, EW, CHS, NCH):
    mesh = plsc.VectorSubcoreMesh(core_axis_name="c", subcore_axis_name="s",
                                  num_cores=NC, num_subcores=NS)

    @functools.partial(
        pl.kernel,
        out_type=jax.ShapeDtypeStruct((E, PW), F32),
        mesh=mesh,
        scratch_types=[pltpu.VMEM((EW,), jnp.int32),
                       pltpu.VMEM((EW,), jnp.int32),
                       pltpu.VMEM((CHS, PW), F32)],
    )
    def sc_gather(t_h, u_h, row_h, col_h, y_o, rowv, colv, ybuf):
        wid = lax.axis_index("s") * NC + lax.axis_index("c")
        base = wid * EW
        pltpu.sync_copy(row_h.at[pl.ds(base, EW)], rowv)
        pltpu.sync_copy(col_h.at[pl.ds(base, EW)], colv)

        def body(j, carry):
            off = pl.multiple_of(j * CHS, 8)
            ri = rowv.at[pl.ds(off, CHS)]
            ci = colv.at[pl.ds(off, CHS)]
            pltpu.sync_copy(t_h.at[ri], ybuf)
            pltpu.sync_copy(u_h.at[ci], ybuf, add=True)
            obase = pl.multiple_of(base + j * CHS, 8)
            pltpu.sync_copy(ybuf, y_o.at[pl.ds(obase, CHS)])
            return carry

        lax.fori_loop(0, NCH, body, 0)

    return sc_gather


@functools.lru_cache(maxsize=None)
def _build_sc_scatter(E, N, EW, CHS, NCH):
    mesh = plsc.VectorSubcoreMesh(core_axis_name="c", subcore_axis_name="s",
                                  num_cores=NC, num_subcores=NS)
    NRA = (N // NS + 7) // 8 * 8
    NRL = N - NRA * (NS - 1)
    assert NRL > 0 and NRL % 8 == 0

    @functools.partial(
        pl.kernel,
        out_type=jax.ShapeDtypeStruct((NC, N, PW), F32),
        mesh=mesh,
        scratch_types=[pltpu.VMEM((NCH, CHS), jnp.int32),
                       pltpu.VMEM((CHS, PW), F32),
                       pltpu.VMEM_SHARED((N, PW), F32)],
    )
    def sc_scatter(e2_h, col3_h, z_h, p2_o, colv, ebuf, acc):
        cid = lax.axis_index("c")
        sid = lax.axis_index("s")
        wid = sid * NC + cid
        base = wid * EW
        pltpu.sync_copy(col3_h.at[wid], colv)
        r0 = pl.multiple_of(sid * NRA, 8)

        @pl.when(sid < NS - 1)
        def _():
            pltpu.sync_copy(z_h.at[pl.ds(r0, NRA)], acc.at[pl.ds(r0, NRA)])

        @pl.when(sid == NS - 1)
        def _():
            pltpu.sync_copy(z_h.at[pl.ds(r0, NRL)], acc.at[pl.ds(r0, NRL)])

        plsc.subcore_barrier()

        def body(j, carry):
            off = pl.multiple_of(base + j * CHS, 8)
            ci = colv.at[j]
            pltpu.sync_copy(e2_h.at[pl.ds(off, CHS)], ebuf)
            pltpu.sync_copy(ebuf, acc.at[ci], add=True)
            return carry

        lax.fori_loop(0, NCH, body, 0)
        plsc.subcore_barrier()

        @pl.when(sid < NS - 1)
        def _():
            pltpu.sync_copy(acc.at[pl.ds(r0, NRA)],
                            p2_o.at[cid, pl.ds(r0, NRA)])

        @pl.when(sid == NS - 1)
        def _():
            pltpu.sync_copy(acc.at[pl.ds(r0, NRL)],
                            p2_o.at[cid, pl.ds(r0, NRL)])

    return sc_scatter


def kernel(inputs, hidden, edges, node_masks, send_edges, recv_edges,
           edge2node_inds, emb_w, emb_b, edge_w1, edge_b1, edge_w2, edge_b2,
           node_w1, node_b1, node_w2, node_b2, coord_w1, coord_b1, coord_w2,
           vel_w1, vel_b1, vel_w2, vel_b2):
    N = inputs.shape[1]
    E = send_edges.shape[0]
    H = emb_w.shape[1]
    L = edge_w1.shape[0]
    BN = 2000
    BE = 2560
    EW = E // NW
    CHS = 80
    NCH = EW // CHS

    x = inputs[0]
    xp = jnp.pad(x[:, :3], ((0, 0), (0, DW - 3)))
    xv = jnp.pad(x[:, 3:6], ((0, 0), (0, DW - 3)))
    row = send_edges.astype(jnp.int32)
    col = recv_edges.astype(jnp.int32)
    col3 = col.reshape(NW, NCH, CHS)
    zp = jnp.zeros((N, PW), F32)

    embw = emb_w.reshape(1, H)
    embb = emb_b.reshape(1, H)
    w1a = [edge_w1[l, :H] for l in range(L)]
    w1b = [edge_w1[l, H:2 * H] for l in range(L)]
    w1c = [edge_w1[l, 2 * H:2 * H + 1] for l in range(L)]
    b1 = [edge_b1[l].reshape(1, H) for l in range(L)]
    w2 = [edge_w2[l] for l in range(L)]
    b2 = [edge_b2[l].reshape(1, H) for l in range(L)]
    cw1 = [coord_w1[l] for l in range(L)]
    cb1 = [coord_b1[l].reshape(1, H) for l in range(L)]
    cw2r = [coord_w2[l].reshape(1, H) for l in range(L)]
    vw1 = [vel_w1[l] for l in range(L)]
    vb1 = [vel_b1[l].reshape(1, H) for l in range(L)]
    vw2r = [vel_w2[l].reshape(1, H) for l in range(L)]
    vb2 = [vel_b2[l].reshape(1, 1) for l in range(L)]
    nw1a = [node_w1[l, :H] for l in range(L)]
    nw1b = [node_w1[l, H:] for l in range(L)]
    nb1 = [node_b1[l].reshape(1, H) for l in range(L)]
    nw2 = [node_w2[l] for l in range(L)]
    nb2 = [node_b2[l].reshape(1, H) for l in range(L)]

    sc_gather = _build_sc_gather(E, N, EW, CHS, NCH)
    sc_scatter = _build_sc_scatter(E, N, EW, CHS, NCH)

    h, T, U = _tc_init(xp, xv, embw, embb, w1a[0], w1b[0], N, H, BN)
    pos, vel = xp, xv
    for l in range(L):
        Y = sc_gather(T, U, row, col)
        E2 = _tc_edge(Y, w1c[l], b1[l], w2[l], b2[l], cw1[l], cb1[l],
                      cw2r[l], E, H, BE)
        p2 = sc_scatter(E2, col3, zp)
        nxt = (w1a[l + 1], w1b[l + 1]) if l + 1 < L else None
        outs = _tc_node(h, pos, vel, p2, vw1[l], vb1[l], vw2r[l], vb2[l],
                        nw1a[l], nw1b[l], nb1[l], nw2[l], nb2[l], nxt,
                        N, H, BN)
        if nxt is not None:
            h, pos, vel, T, U = outs
        else:
            h, pos, vel = outs

    return jnp.concatenate([pos[:, :3], vel[:, :3]], axis=-1)[None]


# speedup vs baseline: 6.0800x; 1.3470x over previous
import functools

import jax
import jax.numpy as jnp
from jax import lax
from jax.experimental import pallas as pl
from jax.experimental.pallas import tpu as pltpu
from jax.experimental.pallas import tpu_sc as plsc

F32 = jnp.float32
NC = 2
NS = 16
NW = NC * NS
DW = 16
PW = 128


def _silu(x):
    return x * jax.nn.sigmoid(x)


def _full_spec(shape):
    nd = len(shape)
    return pl.BlockSpec(shape, lambda i, _nd=nd: (0,) * _nd)


def _init_body(xp, xv, embw, embb, w1a, w1b, h_o, t_o, u_o):
    v = xv[...]
    p = xp[...]
    nrm = jnp.sqrt(jnp.sum(v * v, axis=-1, keepdims=True))
    h = nrm * embw[...] + embb[...]
    h_o[...] = h
    z = jnp.zeros((p.shape[0], PW - h.shape[1] - DW), F32)
    ga = jnp.dot(h, w1a[...], preferred_element_type=F32)
    gb = jnp.dot(h, w1b[...], preferred_element_type=F32)
    t_o[...] = jnp.concatenate([ga, p, z], axis=-1)
    u_o[...] = jnp.concatenate([gb, -p, z], axis=-1)


def _tc_init(xp, xv, embw, embb, w1a, w1b, N, H, BN):
    grid = N // BN
    bs_n16 = pl.BlockSpec((BN, DW), lambda i: (i, 0))
    bs_nh = pl.BlockSpec((BN, H), lambda i: (i, 0))
    bs_np = pl.BlockSpec((BN, PW), lambda i: (i, 0))
    return pl.pallas_call(
        _init_body,
        grid=(grid,),
        in_specs=[bs_n16, bs_n16, _full_spec(embw.shape), _full_spec(embb.shape),
                  _full_spec(w1a.shape), _full_spec(w1b.shape)],
        out_specs=[bs_nh, bs_np, bs_np],
        out_shape=[jax.ShapeDtypeStruct((N, H), F32),
                   jax.ShapeDtypeStruct((N, PW), F32),
                   jax.ShapeDtypeStruct((N, PW), F32)],
    )(xp, xv, embw, embb, w1a, w1b)


def _edge_body(H, y, w1c, b1, w2, b2, cw1, cb1, cw2r, e2_o):
    Y = y[...]
    X = Y[:, :H]
    D = Y[:, H:H + DW]
    lane = lax.broadcasted_iota(jnp.int32, D.shape, 1)
    radial = jnp.sum(jnp.where(lane < 3, D * D, 0.0), axis=-1, keepdims=True)
    x1 = X + radial * w1c[...] + b1[...]
    e1 = _silu(x1)
    ef = _silu(jnp.dot(e1, w2[...], preferred_element_type=F32) + b2[...])
    c = _silu(jnp.dot(ef, cw1[...], preferred_element_type=F32) + cb1[...])
    t = jnp.sum(c * cw2r[...], axis=-1, keepdims=True)
    trans = jnp.clip(D * t, -100.0, 100.0)
    s = jnp.where(lane == 3, 1.0, trans)
    z = jnp.zeros((Y.shape[0], PW - H - DW), F32)
    e2_o[...] = jnp.concatenate([ef, s, z], axis=-1)


def _tc_edge(y, w1c, b1, w2, b2, cw1, cb1, cw2r, E, H, BE):
    grid = E // BE
    bs_ep = pl.BlockSpec((BE, PW), lambda i: (i, 0))
    return pl.pallas_call(
        functools.partial(_edge_body, H),
        grid=(grid,),
        in_specs=[bs_ep] + [_full_spec(w.shape)
                            for w in (w1c, b1, w2, b2, cw1, cb1, cw2r)],
        out_specs=[bs_ep],
        out_shape=[jax.ShapeDtypeStruct((E, PW), F32)],
    )(y, w1c, b1, w2, b2, cw1, cb1, cw2r)[0]


def _node_body(emit_next, H, *refs):
    if emit_next:
        (h, pos, vel, p2, vw1, vb1, vw2r, vb2, nw1a, nw1b, nb1, nw2, nb2,
         xa, xb, h_o, pos_o, vel_o, t_o, u_o) = refs
    else:
        (h, pos, vel, p2, vw1, vb1, vw2r, vb2, nw1a, nw1b, nb1, nw2, nb2,
         h_o, pos_o, vel_o) = refs
    hh = h[...]
    P = p2[0] + p2[1]
    agg_e = P[:, :H]
    ag = P[:, H:H + DW]
    lane = lax.broadcasted_iota(jnp.int32, ag.shape, 1)
    cnt = ag[:, 3:4]
    new_vel = jnp.where(lane < 3, ag, 0.0) / jnp.maximum(cnt, 1.0)
    hv = _silu(jnp.dot(hh, vw1[...], preferred_element_type=F32) + vb1[...])
    mv = jnp.sum(hv * vw2r[...], axis=-1, keepdims=True) + vb2[...]
    new_vel = new_vel + mv * vel[...]
    pos2 = pos[...] + new_vel
    pos_o[...] = pos2
    vel_o[...] = new_vel
    pre = _silu(jnp.dot(hh, nw1a[...], preferred_element_type=F32)
                + jnp.dot(agg_e, nw1b[...], preferred_element_type=F32)
                + nb1[...])
    h2 = jnp.dot(pre, nw2[...], preferred_element_type=F32) + nb2[...] + hh
    h_o[...] = h2
    if emit_next:
        z = jnp.zeros((hh.shape[0], PW - H - DW), F32)
        ga = jnp.dot(h2, xa[...], preferred_element_type=F32)
        gb = jnp.dot(h2, xb[...], preferred_element_type=F32)
        t_o[...] = jnp.concatenate([ga, pos2, z], axis=-1)
        u_o[...] = jnp.concatenate([gb, -pos2, z], axis=-1)


def _tc_node(h, pos, vel, p2, vw1, vb1, vw2r, vb2,
             nw1a, nw1b, nb1, nw2, nb2, nxt, N, H, BN):
    grid = N // BN
    bs_nh = pl.BlockSpec((BN, H), lambda i: (i, 0))
    bs_n16 = pl.BlockSpec((BN, DW), lambda i: (i, 0))
    bs_np = pl.BlockSpec((BN, PW), lambda i: (i, 0))
    bs_p2 = pl.BlockSpec((NC, BN, PW), lambda i: (0, i, 0))
    ws = (vw1, vb1, vw2r, vb2, nw1a, nw1b, nb1, nw2, nb2)
    emit_next = nxt is not None
    ins = [h, pos, vel, p2, *ws]
    in_specs = [bs_nh, bs_n16, bs_n16, bs_p2] + [_full_spec(w.shape)
                                                 for w in ws]
    out_specs = [bs_nh, bs_n16, bs_n16]
    out_shape = [jax.ShapeDtypeStruct((N, H), F32),
                 jax.ShapeDtypeStruct((N, DW), F32),
                 jax.ShapeDtypeStruct((N, DW), F32)]
    if emit_next:
        ins += [nxt[0], nxt[1]]
        in_specs += [_full_spec(nxt[0].shape), _full_spec(nxt[1].shape)]
        out_specs += [bs_np, bs_np]
        out_shape += [jax.ShapeDtypeStruct((N, PW), F32),
                      jax.ShapeDtypeStruct((N, PW), F32)]
    return pl.pallas_call(
        functools.partial(_node_body, emit_next, H),
        grid=(grid,),
        in_specs=in_specs,
        out_specs=out_specs,
        out_shape=out_shape,
    )(*ins)


@functools.lru_cache(maxsize=None)
def _build_sc_gather(E, N, EW, CHS, NCH):
    mesh = plsc.VectorSubcoreMesh(core_axis_name="c", subcore_axis_name="s",
                                  num_cores=NC, num_subcores=NS)

    NB = 5
    assert NCH % NB == 0
    NG = NCH // NB

    @functools.partial(
        pl.kernel,
        out_type=jax.ShapeDtypeStruct((E, PW), F32),
        mesh=mesh,
        scratch_types=[pltpu.VMEM((EW,), jnp.int32),
                       pltpu.VMEM((EW,), jnp.int32),
                       pltpu.VMEM((NB, CHS, PW), F32),
                       pltpu.SemaphoreType.DMA,
                       pltpu.SemaphoreType.DMA,
                       pltpu.SemaphoreType.DMA],
    )
    def sc_gather(t_h, u_h, row_h, col_h, y_o, rowv, colv, ybuf,
                  sem_a, sem_b, sem_w):
        wid = lax.axis_index("s") * NC + lax.axis_index("c")
        base = wid * EW
        pltpu.sync_copy(row_h.at[pl.ds(base, EW)], rowv)
        pltpu.sync_copy(col_h.at[pl.ds(base, EW)], colv)

        def outer(g, carry):
            offs = [pl.multiple_of((g * NB + b) * CHS, 8) for b in range(NB)]
            descs = []
            for b in range(NB):
                @pl.when(g > 0)
                def _(b=b):
                    pltpu.make_async_copy(
                        ybuf.at[b], y_o.at[pl.ds(0, CHS)], sem_w).wait()
                descs.append(pltpu.async_copy(
                    t_h.at[rowv.at[pl.ds(offs[b], CHS)]], ybuf.at[b], sem_a))
            descs2 = []
            for b in range(NB):
                descs[b].wait()
                descs2.append(pltpu.async_copy(
                    u_h.at[colv.at[pl.ds(offs[b], CHS)]], ybuf.at[b], sem_b,
                    add=True))
            for b in range(NB):
                descs2[b].wait()
                obase = pl.multiple_of(base + (g * NB + b) * CHS, 8)
                pltpu.async_copy(ybuf.at[b], y_o.at[pl.ds(obase, CHS)], sem_w)
            return carry

        lax.fori_loop(0, NG, outer, 0)
        for b in range(NB):
            pltpu.make_async_copy(
                ybuf.at[b], y_o.at[pl.ds(0, CHS)], sem_w).wait()

    return sc_gather


@functools.lru_cache(maxsize=None)
def _build_sc_scatter(E, N, EW, CHS, NCH):
    mesh = plsc.VectorSubcoreMesh(core_axis_name="c", subcore_axis_name="s",
                                  num_cores=NC, num_subcores=NS)
    NRA = (N // NS + 7) // 8 * 8
    NRL = N - NRA * (NS - 1)
    assert NRL > 0 and NRL % 8 == 0

    NB = 2
    assert NCH % NB == 0
    NG = NCH // NB

    @functools.partial(
        pl.kernel,
        out_type=jax.ShapeDtypeStruct((NC, N, PW), F32),
        mesh=mesh,
        scratch_types=[pltpu.VMEM((NB, CHS), jnp.int32),
                       pltpu.VMEM((NB, CHS, PW), F32),
                       pltpu.VMEM_SHARED((N, PW), F32),
                       pltpu.SemaphoreType.DMA,
                       pltpu.SemaphoreType.DMA],
    )
    def sc_scatter(e2_h, col3_h, z_h, p2_o, idxb, ebuf, acc, sem_r, sem_s):
        cid = lax.axis_index("c")
        sid = lax.axis_index("s")
        wid = sid * NC + cid
        base = wid * EW
        r0 = pl.multiple_of(sid * NRA, 8)

        @pl.when(sid < NS - 1)
        def _():
            pltpu.sync_copy(z_h.at[pl.ds(r0, NRA)], acc.at[pl.ds(r0, NRA)])

        @pl.when(sid == NS - 1)
        def _():
            pltpu.sync_copy(z_h.at[pl.ds(r0, NRL)], acc.at[pl.ds(r0, NRL)])

        plsc.subcore_barrier()

        def outer(g, carry):
            descs = []
            for b in range(NB):
                j = g * NB + b
                off = pl.multiple_of(base + j * CHS, 8)

                @pl.when(g > 0)
                def _(b=b):
                    pltpu.make_async_copy(
                        ebuf.at[b], acc.at[idxb.at[b]], sem_s).wait()
                descs.append((
                    pltpu.async_copy(e2_h.at[pl.ds(off, CHS)], ebuf.at[b],
                                     sem_r),
                    pltpu.async_copy(col3_h.at[wid, j], idxb.at[b], sem_r)))
            for b in range(NB):
                descs[b][0].wait()
                descs[b][1].wait()
                pltpu.async_copy(ebuf.at[b], acc.at[idxb.at[b]], sem_s,
                                 add=True)
            return carry

        lax.fori_loop(0, NG, outer, 0)
        for b in range(NB):
            pltpu.make_async_copy(
                ebuf.at[b], acc.at[idxb.at[b]], sem_s).wait()
        plsc.subcore_barrier()

        @pl.when(sid < NS - 1)
        def _():
            pltpu.sync_copy(acc.at[pl.ds(r0, NRA)],
                            p2_o.at[cid, pl.ds(r0, NRA)])

        @pl.when(sid == NS - 1)
        def _():
            pltpu.sync_copy(acc.at[pl.ds(r0, NRL)],
                            p2_o.at[cid, pl.ds(r0, NRL)])

    return sc_scatter


def kernel(inputs, hidden, edges, node_masks, send_edges, recv_edges,
           edge2node_inds, emb_w, emb_b, edge_w1, edge_b1, edge_w2, edge_b2,
           node_w1, node_b1, node_w2, node_b2, coord_w1, coord_b1, coord_w2,
           vel_w1, vel_b1, vel_w2, vel_b2):
    N = inputs.shape[1]
    E = send_edges.shape[0]
    H = emb_w.shape[1]
    L = edge_w1.shape[0]
    BN = 2000
    BE = 2560
    EW = E // NW
    CHS = 80
    NCH = EW // CHS

    x = inputs[0]
    xp = jnp.pad(x[:, :3], ((0, 0), (0, DW - 3)))
    xv = jnp.pad(x[:, 3:6], ((0, 0), (0, DW - 3)))
    row = send_edges.astype(jnp.int32)
    col = recv_edges.astype(jnp.int32)
    col3 = col.reshape(NW, EW // 40, 40)
    zp = jnp.zeros((N, PW), F32)

    embw = emb_w.reshape(1, H)
    embb = emb_b.reshape(1, H)
    w1a = [edge_w1[l, :H] for l in range(L)]
    w1b = [edge_w1[l, H:2 * H] for l in range(L)]
    w1c = [edge_w1[l, 2 * H:2 * H + 1] for l in range(L)]
    b1 = [edge_b1[l].reshape(1, H) for l in range(L)]
    w2 = [edge_w2[l] for l in range(L)]
    b2 = [edge_b2[l].reshape(1, H) for l in range(L)]
    cw1 = [coord_w1[l] for l in range(L)]
    cb1 = [coord_b1[l].reshape(1, H) for l in range(L)]
    cw2r = [coord_w2[l].reshape(1, H) for l in range(L)]
    vw1 = [vel_w1[l] for l in range(L)]
    vb1 = [vel_b1[l].reshape(1, H) for l in range(L)]
    vw2r = [vel_w2[l].reshape(1, H) for l in range(L)]
    vb2 = [vel_b2[l].reshape(1, 1) for l in range(L)]
    nw1a = [node_w1[l, :H] for l in range(L)]
    nw1b = [node_w1[l, H:] for l in range(L)]
    nb1 = [node_b1[l].reshape(1, H) for l in range(L)]
    nw2 = [node_w2[l] for l in range(L)]
    nb2 = [node_b2[l].reshape(1, H) for l in range(L)]

    CHS2 = 40
    NCH2 = EW // CHS2
    sc_gather = _build_sc_gather(E, N, EW, CHS, NCH)
    sc_scatter = _build_sc_scatter(E, N, EW, CHS2, NCH2)

    h, T, U = _tc_init(xp, xv, embw, embb, w1a[0], w1b[0], N, H, BN)
    pos, vel = xp, xv
    for l in range(L):
        Y = sc_gather(T, U, row, col)
        E2 = _tc_edge(Y, w1c[l], b1[l], w2[l], b2[l], cw1[l], cb1[l],
                      cw2r[l], E, H, BE)
        p2 = sc_scatter(E2, col3, zp)
        nxt = (w1a[l + 1], w1b[l + 1]) if l + 1 < L else None
        outs = _tc_node(h, pos, vel, p2, vw1[l], vb1[l], vw2r[l], vb2[l],
                        nw1a[l], nw1b[l], nb1[l], nw2[l], nb2[l], nxt,
                        N, H, BN)
        if nxt is not None:
            h, pos, vel, T, U = outs
        else:
            h, pos, vel = outs

    return jnp.concatenate([pos[:, :3], vel[:, :3]], axis=-1)[None]


# speedup vs baseline: 7.2107x; 1.1860x over previous
import functools

import jax
import jax.numpy as jnp
from jax import lax
from jax.experimental import pallas as pl
from jax.experimental.pallas import tpu as pltpu
from jax.experimental.pallas import tpu_sc as plsc

F32 = jnp.float32
BF = jnp.bfloat16
NC = 2
NS = 16
NW = NC * NS
DW = 16
PW = 128


def _silu(x):
    return x / (1.0 + jnp.exp(-x))


def _full_spec(shape):
    nd = len(shape)
    return pl.BlockSpec(shape, lambda i, _nd=nd: (0,) * _nd)


def _init_body(xp, xv, embw, embb, w1a, w1b, h_o, t_o, u_o):
    v = xv[...]
    p = xp[...]
    nrm = jnp.sqrt(jnp.sum(v * v, axis=-1, keepdims=True))
    h = nrm * embw[...] + embb[...]
    h_o[...] = h
    z = jnp.zeros((p.shape[0], PW - h.shape[1] - DW), F32)
    ga = jnp.dot(h, w1a[...], preferred_element_type=F32)
    gb = jnp.dot(h, w1b[...], preferred_element_type=F32)
    t_o[...] = jnp.concatenate([ga, p, z], axis=-1)
    u_o[...] = jnp.concatenate([gb, -p, z], axis=-1)


def _tc_init(xp, xv, embw, embb, w1a, w1b, N, H, BN):
    grid = N // BN
    bs_n16 = pl.BlockSpec((BN, DW), lambda i: (i, 0))
    bs_nh = pl.BlockSpec((BN, H), lambda i: (i, 0))
    bs_np = pl.BlockSpec((BN, PW), lambda i: (i, 0))
    return pl.pallas_call(
        _init_body,
        grid=(grid,),
        in_specs=[bs_n16, bs_n16, _full_spec(embw.shape), _full_spec(embb.shape),
                  _full_spec(w1a.shape), _full_spec(w1b.shape)],
        out_specs=[bs_nh, bs_np, bs_np],
        out_shape=[jax.ShapeDtypeStruct((N, H), F32),
                   jax.ShapeDtypeStruct((N, PW), F32),
                   jax.ShapeDtypeStruct((N, PW), F32)],
    )(xp, xv, embw, embb, w1a, w1b)


def _edge_body(H, y, w1c, b1, w2p, b2p, cw1p, cb1, cw2r, e2_o):
    Y = y[...]
    lane = lax.broadcasted_iota(jnp.int32, Y.shape, 1)
    hi = lane >= H
    radial = jnp.sum(jnp.where(hi, Y * Y, 0.0), axis=-1, keepdims=True)
    X = Y[:, :H]
    x1 = X + radial * w1c[...] + b1[...]
    e1 = _silu(x1)
    ef = _silu(jnp.dot(e1, w2p[...], preferred_element_type=F32) + b2p[...])
    c = _silu(jnp.dot(ef, cw1p[...], preferred_element_type=F32) + cb1[...])
    t = jnp.sum(c * cw2r[...], axis=-1, keepdims=True)
    s = jnp.where(hi, jnp.clip(Y * t, -100.0, 100.0), 0.0)
    s = jnp.where(lane == H + 3, 1.0, s)
    e2_o[...] = ef + s


def _tc_edge(y, w1c, b1, w2p, b2p, cw1p, cb1, cw2r, E, H, BE):
    grid = E // BE
    bs_ep = pl.BlockSpec((BE, PW), lambda i: (i, 0))
    return pl.pallas_call(
        functools.partial(_edge_body, H),
        grid=(grid,),
        in_specs=[bs_ep] + [_full_spec(w.shape)
                            for w in (w1c, b1, w2p, b2p, cw1p, cb1, cw2r)],
        out_specs=[bs_ep],
        out_shape=[jax.ShapeDtypeStruct((E, PW), F32)],
    )(y, w1c, b1, w2p, b2p, cw1p, cb1, cw2r)[0]


def _node_body(emit_next, H, *refs):
    if emit_next:
        (h, pos, vel, p2, vw1, vb1, vw2r, vb2, nw1a, nw1b, nb1, nw2, nb2,
         xa, xb, h_o, pos_o, vel_o, t_o, u_o) = refs
    else:
        (h, pos, vel, p2, vw1, vb1, vw2r, vb2, nw1a, nw1b, nb1, nw2, nb2,
         h_o, pos_o, vel_o) = refs
    hh = h[...]
    P = p2[0] + p2[1]
    agg_e = P[:, :H]
    ag = P[:, H:H + DW]
    lane = lax.broadcasted_iota(jnp.int32, ag.shape, 1)
    cnt = ag[:, 3:4]
    new_vel = jnp.where(lane < 3, ag, 0.0) / jnp.maximum(cnt, 1.0)
    hv = _silu(jnp.dot(hh, vw1[...], preferred_element_type=F32) + vb1[...])
    mv = jnp.sum(hv * vw2r[...], axis=-1, keepdims=True) + vb2[...]
    new_vel = new_vel + mv * vel[...]
    pos2 = pos[...] + new_vel
    pos_o[...] = pos2
    vel_o[...] = new_vel
    pre = _silu(jnp.dot(hh, nw1a[...], preferred_element_type=F32)
                + jnp.dot(agg_e, nw1b[...], preferred_element_type=F32)
                + nb1[...])
    h2 = jnp.dot(pre, nw2[...], preferred_element_type=F32) + nb2[...] + hh
    h_o[...] = h2
    if emit_next:
        z = jnp.zeros((hh.shape[0], PW - H - DW), F32)
        ga = jnp.dot(h2, xa[...], preferred_element_type=F32)
        gb = jnp.dot(h2, xb[...], preferred_element_type=F32)
        t_o[...] = jnp.concatenate([ga, pos2, z], axis=-1)
        u_o[...] = jnp.concatenate([gb, -pos2, z], axis=-1)


def _tc_node(h, pos, vel, p2, vw1, vb1, vw2r, vb2,
             nw1a, nw1b, nb1, nw2, nb2, nxt, N, H, BN):
    grid = N // BN
    bs_nh = pl.BlockSpec((BN, H), lambda i: (i, 0))
    bs_n16 = pl.BlockSpec((BN, DW), lambda i: (i, 0))
    bs_np = pl.BlockSpec((BN, PW), lambda i: (i, 0))
    bs_p2 = pl.BlockSpec((NC, BN, PW), lambda i: (0, i, 0))
    ws = (vw1, vb1, vw2r, vb2, nw1a, nw1b, nb1, nw2, nb2)
    emit_next = nxt is not None
    ins = [h, pos, vel, p2, *ws]
    in_specs = [bs_nh, bs_n16, bs_n16, bs_p2] + [_full_spec(w.shape)
                                                 for w in ws]
    out_specs = [bs_nh, bs_n16, bs_n16]
    out_shape = [jax.ShapeDtypeStruct((N, H), F32),
                 jax.ShapeDtypeStruct((N, DW), F32),
                 jax.ShapeDtypeStruct((N, DW), F32)]
    if emit_next:
        ins += [nxt[0], nxt[1]]
        in_specs += [_full_spec(nxt[0].shape), _full_spec(nxt[1].shape)]
        out_specs += [bs_np, bs_np]
        out_shape += [jax.ShapeDtypeStruct((N, PW), F32),
                      jax.ShapeDtypeStruct((N, PW), F32)]
    return pl.pallas_call(
        functools.partial(_node_body, emit_next, H),
        grid=(grid,),
        in_specs=in_specs,
        out_specs=out_specs,
        out_shape=out_shape,
    )(*ins)


@functools.lru_cache(maxsize=None)
def _build_sc_gather(E, N, EW, CHS, NCH):
    mesh = plsc.VectorSubcoreMesh(core_axis_name="c", subcore_axis_name="s",
                                  num_cores=NC, num_subcores=NS)

    NB = 5
    assert NCH % NB == 0
    NG = NCH // NB

    @functools.partial(
        pl.kernel,
        out_type=jax.ShapeDtypeStruct((E, PW), F32),
        mesh=mesh,
        scratch_types=[pltpu.VMEM((EW,), jnp.int32),
                       pltpu.VMEM((EW,), jnp.int32),
                       pltpu.VMEM((NB, CHS, PW), F32),
                       pltpu.SemaphoreType.DMA,
                       pltpu.SemaphoreType.DMA,
                       pltpu.SemaphoreType.DMA],
    )
    def sc_gather(t_h, u_h, row_h, col_h, y_o, rowv, colv, ybuf,
                  sem_a, sem_b, sem_w):
        wid = lax.axis_index("s") * NC + lax.axis_index("c")
        base = wid * EW
        pltpu.sync_copy(row_h.at[pl.ds(base, EW)], rowv)
        pltpu.sync_copy(col_h.at[pl.ds(base, EW)], colv)

        def outer(g, carry):
            offs = [pl.multiple_of((g * NB + b) * CHS, 8) for b in range(NB)]
            descs = []
            for b in range(NB):
                @pl.when(g > 0)
                def _(b=b):
                    pltpu.make_async_copy(
                        ybuf.at[b], y_o.at[pl.ds(0, CHS)], sem_w).wait()
                descs.append(pltpu.async_copy(
                    t_h.at[rowv.at[pl.ds(offs[b], CHS)]], ybuf.at[b], sem_a))
            descs2 = []
            for b in range(NB):
                descs[b].wait()
                descs2.append(pltpu.async_copy(
                    u_h.at[colv.at[pl.ds(offs[b], CHS)]], ybuf.at[b], sem_b,
                    add=True))
            for b in range(NB):
                descs2[b].wait()
                obase = pl.multiple_of(base + (g * NB + b) * CHS, 8)
                pltpu.async_copy(ybuf.at[b], y_o.at[pl.ds(obase, CHS)], sem_w)
            return carry

        lax.fori_loop(0, NG, outer, 0)
        for b in range(NB):
            pltpu.make_async_copy(
                ybuf.at[b], y_o.at[pl.ds(0, CHS)], sem_w).wait()

    return sc_gather


@functools.lru_cache(maxsize=None)
def _build_sc_scatter(E, N, EW, CHS, NCH):
    mesh = plsc.VectorSubcoreMesh(core_axis_name="c", subcore_axis_name="s",
                                  num_cores=NC, num_subcores=NS)
    NRA = (N // NS + 7) // 8 * 8
    NRL = N - NRA * (NS - 1)
    assert NRL > 0 and NRL % 8 == 0

    NB = 2
    assert NCH % NB == 0
    NG = NCH // NB

    @functools.partial(
        pl.kernel,
        out_type=jax.ShapeDtypeStruct((NC, N, PW), F32),
        mesh=mesh,
        scratch_types=[pltpu.VMEM((NB, CHS), jnp.int32),
                       pltpu.VMEM((NB, CHS, PW), F32),
                       pltpu.VMEM_SHARED((N, PW), F32),
                       pltpu.SemaphoreType.DMA,
                       pltpu.SemaphoreType.DMA],
    )
    def sc_scatter(e2_h, col3_h, z_h, p2_o, idxb, ebuf, acc, sem_r, sem_s):
        cid = lax.axis_index("c")
        sid = lax.axis_index("s")
        wid = sid * NC + cid
        base = wid * EW
        r0 = pl.multiple_of(sid * NRA, 8)

        @pl.when(sid < NS - 1)
        def _():
            pltpu.sync_copy(z_h.at[pl.ds(r0, NRA)], acc.at[pl.ds(r0, NRA)])

        @pl.when(sid == NS - 1)
        def _():
            pltpu.sync_copy(z_h.at[pl.ds(r0, NRL)], acc.at[pl.ds(r0, NRL)])

        plsc.subcore_barrier()

        def outer(g, carry):
            descs = []
            for b in range(NB):
                j = g * NB + b
                off = pl.multiple_of(base + j * CHS, 8)

                @pl.when(g > 0)
                def _(b=b):
                    pltpu.make_async_copy(
                        ebuf.at[b], acc.at[idxb.at[b]], sem_s).wait()
                descs.append((
                    pltpu.async_copy(e2_h.at[pl.ds(off, CHS)], ebuf.at[b],
                                     sem_r),
                    pltpu.async_copy(col3_h.at[wid, j], idxb.at[b], sem_r)))
            for b in range(NB):
                descs[b][0].wait()
                descs[b][1].wait()
                pltpu.async_copy(ebuf.at[b], acc.at[idxb.at[b]], sem_s,
                                 add=True)
            return carry

        lax.fori_loop(0, NG, outer, 0)
        for b in range(NB):
            pltpu.make_async_copy(
                ebuf.at[b], acc.at[idxb.at[b]], sem_s).wait()
        plsc.subcore_barrier()

        @pl.when(sid < NS - 1)
        def _():
            pltpu.sync_copy(acc.at[pl.ds(r0, NRA)],
                            p2_o.at[cid, pl.ds(r0, NRA)])

        @pl.when(sid == NS - 1)
        def _():
            pltpu.sync_copy(acc.at[pl.ds(r0, NRL)],
                            p2_o.at[cid, pl.ds(r0, NRL)])

    return sc_scatter


def kernel(inputs, hidden, edges, node_masks, send_edges, recv_edges,
           edge2node_inds, emb_w, emb_b, edge_w1, edge_b1, edge_w2, edge_b2,
           node_w1, node_b1, node_w2, node_b2, coord_w1, coord_b1, coord_w2,
           vel_w1, vel_b1, vel_w2, vel_b2):
    N = inputs.shape[1]
    E = send_edges.shape[0]
    H = emb_w.shape[1]
    L = edge_w1.shape[0]
    BN = 2000
    BE = 6400
    EW = E // NW
    CHS = 80
    NCH = EW // CHS

    x = inputs[0]
    xp = jnp.pad(x[:, :3], ((0, 0), (0, DW - 3)))
    xv = jnp.pad(x[:, 3:6], ((0, 0), (0, DW - 3)))
    row = send_edges.astype(jnp.int32)
    col = recv_edges.astype(jnp.int32)
    col3 = col.reshape(NW, EW // 40, 40)
    zp = jnp.zeros((N, PW), F32)

    embw = emb_w.reshape(1, H)
    embb = emb_b.reshape(1, H)
    w1a = [edge_w1[l, :H] for l in range(L)]
    w1b = [edge_w1[l, H:2 * H] for l in range(L)]
    w1c = [edge_w1[l, 2 * H:2 * H + 1] for l in range(L)]
    b1 = [edge_b1[l].reshape(1, H) for l in range(L)]
    w2 = [jnp.pad(edge_w2[l], ((0, 0), (0, PW - H))) for l in range(L)]
    b2 = [jnp.pad(edge_b2[l].reshape(1, H), ((0, 0), (0, PW - H)))
          for l in range(L)]
    cw1 = [jnp.pad(coord_w1[l], ((0, PW - H), (0, 0))) for l in range(L)]
    cb1 = [coord_b1[l].reshape(1, H) for l in range(L)]
    cw2r = [coord_w2[l].reshape(1, H) for l in range(L)]
    vw1 = [vel_w1[l] for l in range(L)]
    vb1 = [vel_b1[l].reshape(1, H) for l in range(L)]
    vw2r = [vel_w2[l].reshape(1, H) for l in range(L)]
    vb2 = [vel_b2[l].reshape(1, 1) for l in range(L)]
    nw1a = [node_w1[l, :H] for l in range(L)]
    nw1b = [node_w1[l, H:] for l in range(L)]
    nb1 = [node_b1[l].reshape(1, H) for l in range(L)]
    nw2 = [node_w2[l] for l in range(L)]
    nb2 = [node_b2[l].reshape(1, H) for l in range(L)]

    CHS2 = 40
    NCH2 = EW // CHS2
    sc_gather = _build_sc_gather(E, N, EW, CHS, NCH)
    sc_scatter = _build_sc_scatter(E, N, EW, CHS2, NCH2)

    h, T, U = _tc_init(xp, xv, embw, embb, w1a[0], w1b[0], N, H, BN)
    pos, vel = xp, xv
    for l in range(L):
        Y = sc_gather(T, U, row, col)
        E2 = _tc_edge(Y, w1c[l], b1[l], w2[l], b2[l], cw1[l], cb1[l],
                      cw2r[l], E, H, BE)
        p2 = sc_scatter(E2, col3, zp)
        nxt = (w1a[l + 1], w1b[l + 1]) if l + 1 < L else None
        outs = _tc_node(h, pos, vel, p2, vw1[l], vb1[l], vw2r[l], vb2[l],
                        nw1a[l], nw1b[l], nb1[l], nw2[l], nb2[l], nxt,
                        N, H, BN)
        if nxt is not None:
            h, pos, vel, T, U = outs
        else:
            h, pos, vel = outs

    return jnp.concatenate([pos[:, :3], vel[:, :3]], axis=-1)[None]


# speedup vs baseline: 8.2547x; 1.1448x over previous
import functools

import jax
import jax.numpy as jnp
from jax import lax
from jax.experimental import pallas as pl
from jax.experimental.pallas import tpu as pltpu
from jax.experimental.pallas import tpu_sc as plsc

F32 = jnp.float32
BF = jnp.bfloat16
NC = 2
NS = 16
NW = NC * NS
DW = 16
PW = 128


def _silu(x):
    return x / (1.0 + jnp.exp(-x))


def _full_spec(shape):
    nd = len(shape)
    return pl.BlockSpec(shape, lambda i, _nd=nd: (0,) * _nd)


def _init_body(xp, xv, embw, embb, w1a, w1b, h_o, t_o, u_o):
    v = xv[...]
    p = xp[...]
    nrm = jnp.sqrt(jnp.sum(v * v, axis=-1, keepdims=True))
    h = nrm * embw[...] + embb[...]
    h_o[...] = h
    z = jnp.zeros((p.shape[0], PW - h.shape[1] - DW), F32)
    ga = jnp.dot(h, w1a[...], preferred_element_type=F32)
    gb = jnp.dot(h, w1b[...], preferred_element_type=F32)
    t_o[...] = jnp.concatenate([ga, p, z], axis=-1)
    u_o[...] = jnp.concatenate([gb, -p, z], axis=-1)


def _tc_init(xp, xv, embw, embb, w1a, w1b, N, H, BN):
    grid = N // BN
    bs_n16 = pl.BlockSpec((BN, DW), lambda i: (i, 0))
    bs_nh = pl.BlockSpec((BN, H), lambda i: (i, 0))
    bs_np = pl.BlockSpec((BN, PW), lambda i: (i, 0))
    return pl.pallas_call(
        _init_body,
        grid=(grid,),
        in_specs=[bs_n16, bs_n16, _full_spec(embw.shape), _full_spec(embb.shape),
                  _full_spec(w1a.shape), _full_spec(w1b.shape)],
        out_specs=[bs_nh, bs_np, bs_np],
        out_shape=[jax.ShapeDtypeStruct((N, H), F32),
                   jax.ShapeDtypeStruct((N, PW), F32),
                   jax.ShapeDtypeStruct((N, PW), F32)],
    )(xp, xv, embw, embb, w1a, w1b)


def _edge_body(H, y, w1c, b1, w2p, b2p, cw1p, cb1, cw2r, e2_o):
    Y = y[...]
    lane = lax.broadcasted_iota(jnp.int32, Y.shape, 1)
    hi = lane >= H
    radial = jnp.sum(jnp.where(hi, Y * Y, 0.0), axis=-1, keepdims=True)
    X = Y[:, :H]
    x1 = X + radial * w1c[...] + b1[...]
    e1 = _silu(x1)
    ef = _silu(jnp.dot(e1, w2p[...], preferred_element_type=F32) + b2p[...])
    c = _silu(jnp.dot(ef, cw1p[...], preferred_element_type=F32) + cb1[...])
    t = jnp.sum(c * cw2r[...], axis=-1, keepdims=True)
    s = jnp.where(hi, jnp.clip(Y * t, -100.0, 100.0), 0.0)
    s = jnp.where(lane == H + 3, 1.0, s)
    e2_o[...] = ef + s


def _tc_edge(y, w1c, b1, w2p, b2p, cw1p, cb1, cw2r, E, H, BE):
    grid = E // BE
    bs_ep = pl.BlockSpec((BE, PW), lambda i: (i, 0))
    return pl.pallas_call(
        functools.partial(_edge_body, H),
        grid=(grid,),
        in_specs=[bs_ep] + [_full_spec(w.shape)
                            for w in (w1c, b1, w2p, b2p, cw1p, cb1, cw2r)],
        out_specs=[bs_ep],
        out_shape=[jax.ShapeDtypeStruct((E, PW), F32)],
    )(y, w1c, b1, w2p, b2p, cw1p, cb1, cw2r)[0]


def _node_body(emit_next, H, *refs):
    if emit_next:
        (h, pos, vel, p2a, p2b, vw1, vb1, vw2r, vb2, nw1a, nw1b, nb1, nw2,
         nb2, xa, xb, h_o, pos_o, vel_o, t_o, u_o) = refs
    else:
        (h, pos, vel, p2a, p2b, vw1, vb1, vw2r, vb2, nw1a, nw1b, nb1, nw2,
         nb2, h_o, pos_o, vel_o) = refs
    hh = h[...]
    P = (p2a[0] + p2a[1]) + (p2b[0] + p2b[1])
    agg_e = P[:, :H]
    ag = P[:, H:H + DW]
    lane = lax.broadcasted_iota(jnp.int32, ag.shape, 1)
    cnt = ag[:, 3:4]
    new_vel = jnp.where(lane < 3, ag, 0.0) / jnp.maximum(cnt, 1.0)
    hv = _silu(jnp.dot(hh, vw1[...], preferred_element_type=F32) + vb1[...])
    mv = jnp.sum(hv * vw2r[...], axis=-1, keepdims=True) + vb2[...]
    new_vel = new_vel + mv * vel[...]
    pos2 = pos[...] + new_vel
    pos_o[...] = pos2
    vel_o[...] = new_vel
    pre = _silu(jnp.dot(hh, nw1a[...], preferred_element_type=F32)
                + jnp.dot(agg_e, nw1b[...], preferred_element_type=F32)
                + nb1[...])
    h2 = jnp.dot(pre, nw2[...], preferred_element_type=F32) + nb2[...] + hh
    h_o[...] = h2
    if emit_next:
        z = jnp.zeros((hh.shape[0], PW - H - DW), F32)
        ga = jnp.dot(h2, xa[...], preferred_element_type=F32)
        gb = jnp.dot(h2, xb[...], preferred_element_type=F32)
        t_o[...] = jnp.concatenate([ga, pos2, z], axis=-1)
        u_o[...] = jnp.concatenate([gb, -pos2, z], axis=-1)


def _tc_node(h, pos, vel, p2a, p2b, vw1, vb1, vw2r, vb2,
             nw1a, nw1b, nb1, nw2, nb2, nxt, N, H, BN):
    grid = N // BN
    bs_nh = pl.BlockSpec((BN, H), lambda i: (i, 0))
    bs_n16 = pl.BlockSpec((BN, DW), lambda i: (i, 0))
    bs_np = pl.BlockSpec((BN, PW), lambda i: (i, 0))
    bs_p2 = pl.BlockSpec((NC, BN, PW), lambda i: (0, i, 0))
    ws = (vw1, vb1, vw2r, vb2, nw1a, nw1b, nb1, nw2, nb2)
    emit_next = nxt is not None
    ins = [h, pos, vel, p2a, p2b, *ws]
    in_specs = [bs_nh, bs_n16, bs_n16, bs_p2, bs_p2] + [_full_spec(w.shape)
                                                        for w in ws]
    out_specs = [bs_nh, bs_n16, bs_n16]
    out_shape = [jax.ShapeDtypeStruct((N, H), F32),
                 jax.ShapeDtypeStruct((N, DW), F32),
                 jax.ShapeDtypeStruct((N, DW), F32)]
    if emit_next:
        ins += [nxt[0], nxt[1]]
        in_specs += [_full_spec(nxt[0].shape), _full_spec(nxt[1].shape)]
        out_specs += [bs_np, bs_np]
        out_shape += [jax.ShapeDtypeStruct((N, PW), F32),
                      jax.ShapeDtypeStruct((N, PW), F32)]
    return pl.pallas_call(
        functools.partial(_node_body, emit_next, H),
        grid=(grid,),
        in_specs=in_specs,
        out_specs=out_specs,
        out_shape=out_shape,
    )(*ins)


@functools.lru_cache(maxsize=None)
def _build_sc_gather(E, N, EW, CHS, NCH):
    mesh = plsc.VectorSubcoreMesh(core_axis_name="c", subcore_axis_name="s",
                                  num_cores=NC, num_subcores=NS)

    NB = 5
    assert NCH % NB == 0
    NG = NCH // NB

    @functools.partial(
        pl.kernel,
        out_type=jax.ShapeDtypeStruct((E, PW), F32),
        mesh=mesh,
        scratch_types=[pltpu.VMEM((EW,), jnp.int32),
                       pltpu.VMEM((EW,), jnp.int32),
                       pltpu.VMEM((NB, CHS, PW), F32),
                       pltpu.SemaphoreType.DMA,
                       pltpu.SemaphoreType.DMA,
                       pltpu.SemaphoreType.DMA],
    )
    def sc_gather(t_h, u_h, row_h, col_h, y_o, rowv, colv, ybuf,
                  sem_a, sem_b, sem_w):
        wid = lax.axis_index("s") * NC + lax.axis_index("c")
        base = wid * EW
        pltpu.sync_copy(row_h.at[pl.ds(base, EW)], rowv)
        pltpu.sync_copy(col_h.at[pl.ds(base, EW)], colv)

        def outer(g, carry):
            offs = [pl.multiple_of((g * NB + b) * CHS, 8) for b in range(NB)]
            descs = []
            for b in range(NB):
                @pl.when(g > 0)
                def _(b=b):
                    pltpu.make_async_copy(
                        ybuf.at[b], y_o.at[pl.ds(0, CHS)], sem_w).wait()
                descs.append(pltpu.async_copy(
                    t_h.at[rowv.at[pl.ds(offs[b], CHS)]], ybuf.at[b], sem_a))
            descs2 = []
            for b in range(NB):
                descs[b].wait()
                descs2.append(pltpu.async_copy(
                    u_h.at[colv.at[pl.ds(offs[b], CHS)]], ybuf.at[b], sem_b,
                    add=True))
            for b in range(NB):
                descs2[b].wait()
                obase = pl.multiple_of(base + (g * NB + b) * CHS, 8)
                pltpu.async_copy(ybuf.at[b], y_o.at[pl.ds(obase, CHS)], sem_w)
            return carry

        lax.fori_loop(0, NG, outer, 0)
        for b in range(NB):
            pltpu.make_async_copy(
                ybuf.at[b], y_o.at[pl.ds(0, CHS)], sem_w).wait()

    return sc_gather


@functools.lru_cache(maxsize=None)
def _build_sc_scatter(E, N, EW, CHS, NCH):
    mesh = plsc.VectorSubcoreMesh(core_axis_name="c", subcore_axis_name="s",
                                  num_cores=NC, num_subcores=NS)
    NRA = (N // NS + 7) // 8 * 8
    NRL = N - NRA * (NS - 1)
    assert NRL > 0 and NRL % 8 == 0

    NB = 2
    NG = NCH // NB
    NREM = NCH - NG * NB

    @functools.partial(
        pl.kernel,
        out_type=jax.ShapeDtypeStruct((NC, N, PW), F32),
        mesh=mesh,
        scratch_types=[pltpu.VMEM((NCH, CHS), jnp.int32),
                       pltpu.VMEM((NB, CHS, PW), F32),
                       pltpu.VMEM_SHARED((N, PW), F32),
                       pltpu.SemaphoreType.DMA,
                       pltpu.SemaphoreType.DMA],
    )
    def sc_scatter(e2_h, col3_h, z_h, p2_o, colv, ebuf, acc, sem_r, sem_s):
        cid = lax.axis_index("c")
        sid = lax.axis_index("s")
        wid = sid * NC + cid
        base = wid * EW
        pltpu.sync_copy(col3_h.at[wid], colv)
        r0 = pl.multiple_of(sid * NRA, 8)

        @pl.when(sid < NS - 1)
        def _():
            pltpu.sync_copy(z_h.at[pl.ds(r0, NRA)], acc.at[pl.ds(r0, NRA)])

        @pl.when(sid == NS - 1)
        def _():
            pltpu.sync_copy(z_h.at[pl.ds(r0, NRL)], acc.at[pl.ds(r0, NRL)])

        plsc.subcore_barrier()

        def outer(g, carry):
            descs = []
            for b in range(NB):
                j = g * NB + b
                off = pl.multiple_of(base + j * CHS, 8)

                @pl.when(g > 0)
                def _(b=b):
                    pltpu.make_async_copy(
                        ebuf.at[b], acc.at[colv.at[0]], sem_s).wait()
                descs.append(pltpu.async_copy(
                    e2_h.at[pl.ds(off, CHS)], ebuf.at[b], sem_r))
            for b in range(NB):
                j = g * NB + b
                descs[b].wait()
                pltpu.async_copy(ebuf.at[b], acc.at[colv.at[j]], sem_s,
                                 add=True)
            return carry

        lax.fori_loop(0, NG, outer, 0)
        for b in range(NB):
            pltpu.make_async_copy(
                ebuf.at[b], acc.at[colv.at[0]], sem_s).wait()
        for r in range(NREM):
            j = NG * NB + r
            off = pl.multiple_of(base + j * CHS, 8)
            pltpu.sync_copy(e2_h.at[pl.ds(off, CHS)], ebuf.at[0])
            pltpu.sync_copy(ebuf.at[0], acc.at[colv.at[j]], add=True)
        plsc.subcore_barrier()

        @pl.when(sid < NS - 1)
        def _():
            pltpu.sync_copy(acc.at[pl.ds(r0, NRA)],
                            p2_o.at[cid, pl.ds(r0, NRA)])

        @pl.when(sid == NS - 1)
        def _():
            pltpu.sync_copy(acc.at[pl.ds(r0, NRL)],
                            p2_o.at[cid, pl.ds(r0, NRL)])

    return sc_scatter


def kernel(inputs, hidden, edges, node_masks, send_edges, recv_edges,
           edge2node_inds, emb_w, emb_b, edge_w1, edge_b1, edge_w2, edge_b2,
           node_w1, node_b1, node_w2, node_b2, coord_w1, coord_b1, coord_w2,
           vel_w1, vel_b1, vel_w2, vel_b2):
    N = inputs.shape[1]
    E = send_edges.shape[0]
    H = emb_w.shape[1]
    L = edge_w1.shape[0]
    BN = 2000
    BE = 6400
    EH = E // 2
    EW = EH // NW
    CHS = 40
    NCH = EW // CHS

    x = inputs[0]
    xp = jnp.pad(x[:, :3], ((0, 0), (0, DW - 3)))
    xv = jnp.pad(x[:, 3:6], ((0, 0), (0, DW - 3)))
    row = send_edges.astype(jnp.int32)
    col = recv_edges.astype(jnp.int32)
    rows = [row[:EH], row[EH:]]
    cols = [col[:EH], col[EH:]]
    col3s = [c.reshape(NW, EW // 40, 40) for c in cols]
    zp = jnp.zeros((N, PW), F32)

    embw = emb_w.reshape(1, H)
    embb = emb_b.reshape(1, H)
    w1a = [edge_w1[l, :H] for l in range(L)]
    w1b = [edge_w1[l, H:2 * H] for l in range(L)]
    w1c = [edge_w1[l, 2 * H:2 * H + 1] for l in range(L)]
    b1 = [edge_b1[l].reshape(1, H) for l in range(L)]
    w2 = [jnp.pad(edge_w2[l], ((0, 0), (0, PW - H))) for l in range(L)]
    b2 = [jnp.pad(edge_b2[l].reshape(1, H), ((0, 0), (0, PW - H)))
          for l in range(L)]
    cw1 = [jnp.pad(coord_w1[l], ((0, PW - H), (0, 0))) for l in range(L)]
    cb1 = [coord_b1[l].reshape(1, H) for l in range(L)]
    cw2r = [coord_w2[l].reshape(1, H) for l in range(L)]
    vw1 = [vel_w1[l] for l in range(L)]
    vb1 = [vel_b1[l].reshape(1, H) for l in range(L)]
    vw2r = [vel_w2[l].reshape(1, H) for l in range(L)]
    vb2 = [vel_b2[l].reshape(1, 1) for l in range(L)]
    nw1a = [node_w1[l, :H] for l in range(L)]
    nw1b = [node_w1[l, H:] for l in range(L)]
    nb1 = [node_b1[l].reshape(1, H) for l in range(L)]
    nw2 = [node_w2[l] for l in range(L)]
    nb2 = [node_b2[l].reshape(1, H) for l in range(L)]

    CHS2 = 40
    NCH2 = EW // CHS2
    sc_gather = _build_sc_gather(EH, N, EW, CHS, NCH)
    sc_scatter = _build_sc_scatter(EH, N, EW, CHS2, NCH2)

    h, T, U = _tc_init(xp, xv, embw, embb, w1a[0], w1b[0], N, H, BN)
    pos, vel = xp, xv
    for l in range(L):
        Ya = sc_gather(T, U, rows[0], cols[0])
        Yb = sc_gather(T, U, rows[1], cols[1])
        Ea = _tc_edge(Ya, w1c[l], b1[l], w2[l], b2[l], cw1[l], cb1[l],
                      cw2r[l], EH, H, BE)
        p2a = sc_scatter(Ea, col3s[0], zp)
        Eb = _tc_edge(Yb, w1c[l], b1[l], w2[l], b2[l], cw1[l], cb1[l],
                      cw2r[l], EH, H, BE)
        p2b = sc_scatter(Eb, col3s[1], zp)
        nxt = (w1a[l + 1], w1b[l + 1]) if l + 1 < L else None
        outs = _tc_node(h, pos, vel, p2a, p2b, vw1[l], vb1[l], vw2r[l],
                        vb2[l], nw1a[l], nw1b[l], nb1[l], nw2[l], nb2[l],
                        nxt, N, H, BN)
        if nxt is not None:
            h, pos, vel, T, U = outs
        else:
            h, pos, vel = outs

    return jnp.concatenate([pos[:, :3], vel[:, :3]], axis=-1)[None]


# speedup vs baseline: 8.7915x; 1.0650x over previous
import functools

import jax
import jax.numpy as jnp
from jax import lax
from jax.experimental import pallas as pl
from jax.experimental.pallas import tpu as pltpu
from jax.experimental.pallas import tpu_sc as plsc

F32 = jnp.float32
BF = jnp.bfloat16
NC = 2
NS = 16
NW = NC * NS
DW = 16
PW = 128


def _silu(x):
    return x / (1.0 + jnp.exp(-x))


def _full_spec(shape):
    nd = len(shape)
    return pl.BlockSpec(shape, lambda i, _nd=nd: (0,) * _nd)


def _init_body(xp, xv, embw, embb, w1a, w1b, h_o, t_o, u_o):
    v = xv[...]
    p = xp[...]
    nrm = jnp.sqrt(jnp.sum(v * v, axis=-1, keepdims=True))
    h = nrm * embw[...] + embb[...]
    h_o[...] = h
    z = jnp.zeros((p.shape[0], PW - h.shape[1] - DW), F32)
    ga = jnp.dot(h, w1a[...], preferred_element_type=F32)
    gb = jnp.dot(h, w1b[...], preferred_element_type=F32)
    t_o[...] = jnp.concatenate([ga, p, z], axis=-1)
    u_o[...] = jnp.concatenate([gb, -p, z], axis=-1)


def _tc_init(xp, xv, embw, embb, w1a, w1b, N, H, BN):
    grid = N // BN
    bs_n16 = pl.BlockSpec((BN, DW), lambda i: (i, 0))
    bs_nh = pl.BlockSpec((BN, H), lambda i: (i, 0))
    bs_np = pl.BlockSpec((BN, PW), lambda i: (i, 0))
    return pl.pallas_call(
        _init_body,
        grid=(grid,),
        in_specs=[bs_n16, bs_n16, _full_spec(embw.shape), _full_spec(embb.shape),
                  _full_spec(w1a.shape), _full_spec(w1b.shape)],
        out_specs=[bs_nh, bs_np, bs_np],
        out_shape=[jax.ShapeDtypeStruct((N, H), F32),
                   jax.ShapeDtypeStruct((N, PW), F32),
                   jax.ShapeDtypeStruct((N, PW), F32)],
    )(xp, xv, embw, embb, w1a, w1b)


def _edge_body(H, y, w1c, b1, w2p, b2p, cw1p, cb1, cw2r, e2_o):
    Y = y[...]
    lane = lax.broadcasted_iota(jnp.int32, Y.shape, 1)
    hi = lane >= H
    radial = jnp.sum(jnp.where(hi, Y * Y, 0.0), axis=-1, keepdims=True)
    X = Y[:, :H]
    x1 = X + radial * w1c[...] + b1[...]
    e1 = _silu(x1)
    ef = _silu(jnp.dot(e1, w2p[...], preferred_element_type=F32) + b2p[...])
    c = _silu(jnp.dot(ef, cw1p[...], preferred_element_type=F32) + cb1[...])
    t = jnp.sum(c * cw2r[...], axis=-1, keepdims=True)
    s = jnp.where(hi, jnp.clip(Y * t, -100.0, 100.0), 0.0)
    s = jnp.where(lane == H + 3, 1.0, s)
    e2_o[...] = ef + s


def _tc_edge(y, w1c, b1, w2p, b2p, cw1p, cb1, cw2r, E, H, BE):
    grid = E // BE
    bs_ep = pl.BlockSpec((BE, PW), lambda i: (i, 0))
    return pl.pallas_call(
        functools.partial(_edge_body, H),
        grid=(grid,),
        in_specs=[bs_ep] + [_full_spec(w.shape)
                            for w in (w1c, b1, w2p, b2p, cw1p, cb1, cw2r)],
        out_specs=[bs_ep],
        out_shape=[jax.ShapeDtypeStruct((E, PW), F32)],
    )(y, w1c, b1, w2p, b2p, cw1p, cb1, cw2r)[0]


def _node_body(emit_next, H, *refs):
    if emit_next:
        (h, pos, vel, p2a, p2b, vw1, vb1, vw2r, vb2, nw1a, nw1b, nb1, nw2,
         nb2, xa, xb, h_o, pos_o, vel_o, t_o, u_o) = refs
    else:
        (h, pos, vel, p2a, p2b, vw1, vb1, vw2r, vb2, nw1a, nw1b, nb1, nw2,
         nb2, h_o, pos_o, vel_o) = refs
    hh = h[...]
    P = (p2a[0] + p2a[1]) + (p2b[0] + p2b[1])
    agg_e = P[:, :H]
    ag = P[:, H:H + DW]
    lane = lax.broadcasted_iota(jnp.int32, ag.shape, 1)
    cnt = ag[:, 3:4]
    new_vel = jnp.where(lane < 3, ag, 0.0) / jnp.maximum(cnt, 1.0)
    hv = _silu(jnp.dot(hh, vw1[...], preferred_element_type=F32) + vb1[...])
    mv = jnp.sum(hv * vw2r[...], axis=-1, keepdims=True) + vb2[...]
    new_vel = new_vel + mv * vel[...]
    pos2 = pos[...] + new_vel
    pos_o[...] = pos2
    vel_o[...] = new_vel
    pre = _silu(jnp.dot(hh, nw1a[...], preferred_element_type=F32)
                + jnp.dot(agg_e, nw1b[...], preferred_element_type=F32)
                + nb1[...])
    h2 = jnp.dot(pre, nw2[...], preferred_element_type=F32) + nb2[...] + hh
    h_o[...] = h2
    if emit_next:
        z = jnp.zeros((hh.shape[0], PW - H - DW), F32)
        ga = jnp.dot(h2, xa[...], preferred_element_type=F32)
        gb = jnp.dot(h2, xb[...], preferred_element_type=F32)
        t_o[...] = jnp.concatenate([ga, pos2, z], axis=-1)
        u_o[...] = jnp.concatenate([gb, -pos2, z], axis=-1)


def _tc_node(h, pos, vel, p2a, p2b, vw1, vb1, vw2r, vb2,
             nw1a, nw1b, nb1, nw2, nb2, nxt, N, H, BN):
    grid = N // BN
    bs_nh = pl.BlockSpec((BN, H), lambda i: (i, 0))
    bs_n16 = pl.BlockSpec((BN, DW), lambda i: (i, 0))
    bs_np = pl.BlockSpec((BN, PW), lambda i: (i, 0))
    bs_p2 = pl.BlockSpec((NC, BN, PW), lambda i: (0, i, 0))
    ws = (vw1, vb1, vw2r, vb2, nw1a, nw1b, nb1, nw2, nb2)
    emit_next = nxt is not None
    ins = [h, pos, vel, p2a, p2b, *ws]
    in_specs = [bs_nh, bs_n16, bs_n16, bs_p2, bs_p2] + [_full_spec(w.shape)
                                                        for w in ws]
    out_specs = [bs_nh, bs_n16, bs_n16]
    out_shape = [jax.ShapeDtypeStruct((N, H), F32),
                 jax.ShapeDtypeStruct((N, DW), F32),
                 jax.ShapeDtypeStruct((N, DW), F32)]
    if emit_next:
        ins += [nxt[0], nxt[1]]
        in_specs += [_full_spec(nxt[0].shape), _full_spec(nxt[1].shape)]
        out_specs += [bs_np, bs_np]
        out_shape += [jax.ShapeDtypeStruct((N, PW), F32),
                      jax.ShapeDtypeStruct((N, PW), F32)]
    return pl.pallas_call(
        functools.partial(_node_body, emit_next, H),
        grid=(grid,),
        in_specs=in_specs,
        out_specs=out_specs,
        out_shape=out_shape,
    )(*ins)


@functools.lru_cache(maxsize=None)
def _build_sc_gather(E, N, EW, CHS, NCH):
    mesh = plsc.VectorSubcoreMesh(core_axis_name="c", subcore_axis_name="s",
                                  num_cores=NC, num_subcores=NS)

    NB = 5
    assert NCH % NB == 0
    NG = NCH // NB

    @functools.partial(
        pl.kernel,
        out_type=jax.ShapeDtypeStruct((E, PW), F32),
        mesh=mesh,
        scratch_types=[pltpu.VMEM((EW,), jnp.int32),
                       pltpu.VMEM((EW,), jnp.int32),
                       pltpu.VMEM((NB, CHS, PW), F32),
                       pltpu.SemaphoreType.DMA,
                       pltpu.SemaphoreType.DMA,
                       pltpu.SemaphoreType.DMA],
    )
    def sc_gather(t_h, u_h, row_h, col_h, y_o, rowv, colv, ybuf,
                  sem_a, sem_b, sem_w):
        wid = lax.axis_index("s") * NC + lax.axis_index("c")
        base = wid * EW
        pltpu.sync_copy(row_h.at[pl.ds(base, EW)], rowv)
        pltpu.sync_copy(col_h.at[pl.ds(base, EW)], colv)

        def outer(g, carry):
            offs = [pl.multiple_of((g * NB + b) * CHS, 8) for b in range(NB)]
            descs = []
            for b in range(NB):
                @pl.when(g > 0)
                def _(b=b):
                    pltpu.make_async_copy(
                        ybuf.at[b], y_o.at[pl.ds(0, CHS)], sem_w).wait()
                descs.append(pltpu.async_copy(
                    t_h.at[rowv.at[pl.ds(offs[b], CHS)]], ybuf.at[b], sem_a))
            descs2 = []
            for b in range(NB):
                descs[b].wait()
                descs2.append(pltpu.async_copy(
                    u_h.at[colv.at[pl.ds(offs[b], CHS)]], ybuf.at[b], sem_b,
                    add=True))
            for b in range(NB):
                descs2[b].wait()
                obase = pl.multiple_of(base + (g * NB + b) * CHS, 8)
                pltpu.async_copy(ybuf.at[b], y_o.at[pl.ds(obase, CHS)], sem_w)
            return carry

        lax.fori_loop(0, NG, outer, 0)
        for b in range(NB):
            pltpu.make_async_copy(
                ybuf.at[b], y_o.at[pl.ds(0, CHS)], sem_w).wait()

    return sc_gather


@functools.lru_cache(maxsize=None)
def _build_sc_scatter(E, N, EW, CHS, NCH):
    mesh = plsc.VectorSubcoreMesh(core_axis_name="c", subcore_axis_name="s",
                                  num_cores=NC, num_subcores=NS)
    NRA = (N // NS + 7) // 8 * 8
    NRL = N - NRA * (NS - 1)
    assert NRL > 0 and NRL % 8 == 0

    NB = 3
    NG = NCH // NB
    NREM = NCH - NG * NB

    @functools.partial(
        pl.kernel,
        out_type=jax.ShapeDtypeStruct((NC, N, PW), F32),
        mesh=mesh,
        scratch_types=[pltpu.VMEM((NCH, CHS), jnp.int32),
                       pltpu.VMEM((NB, CHS, PW), F32),
                       pltpu.VMEM_SHARED((N, PW), F32),
                       pltpu.SemaphoreType.DMA,
                       pltpu.SemaphoreType.DMA],
    )
    def sc_scatter(e2_h, col3_h, z_h, p2_o, colv, ebuf, acc, sem_r, sem_s):
        cid = lax.axis_index("c")
        sid = lax.axis_index("s")
        wid = sid * NC + cid
        base = wid * EW
        pltpu.sync_copy(col3_h.at[wid], colv)
        r0 = pl.multiple_of(sid * NRA, 8)

        @pl.when(sid < NS - 1)
        def _():
            pltpu.sync_copy(z_h.at[pl.ds(r0, NRA)], acc.at[pl.ds(r0, NRA)])

        @pl.when(sid == NS - 1)
        def _():
            pltpu.sync_copy(z_h.at[pl.ds(r0, NRL)], acc.at[pl.ds(r0, NRL)])

        plsc.subcore_barrier()

        def outer(g, carry):
            descs = []
            for b in range(NB):
                j = g * NB + b
                off = pl.multiple_of(base + j * CHS, 8)

                @pl.when(g > 0)
                def _(b=b):
                    pltpu.make_async_copy(
                        ebuf.at[b], acc.at[colv.at[0]], sem_s).wait()
                descs.append(pltpu.async_copy(
                    e2_h.at[pl.ds(off, CHS)], ebuf.at[b], sem_r))
            for b in range(NB):
                j = g * NB + b
                descs[b].wait()
                pltpu.async_copy(ebuf.at[b], acc.at[colv.at[j]], sem_s,
                                 add=True)
            return carry

        lax.fori_loop(0, NG, outer, 0)
        for b in range(NB):
            pltpu.make_async_copy(
                ebuf.at[b], acc.at[colv.at[0]], sem_s).wait()
        for r in range(NREM):
            j = NG * NB + r
            off = pl.multiple_of(base + j * CHS, 8)
            pltpu.sync_copy(e2_h.at[pl.ds(off, CHS)], ebuf.at[0])
            pltpu.sync_copy(ebuf.at[0], acc.at[colv.at[j]], add=True)
        plsc.subcore_barrier()

        @pl.when(sid < NS - 1)
        def _():
            pltpu.sync_copy(acc.at[pl.ds(r0, NRA)],
                            p2_o.at[cid, pl.ds(r0, NRA)])

        @pl.when(sid == NS - 1)
        def _():
            pltpu.sync_copy(acc.at[pl.ds(r0, NRL)],
                            p2_o.at[cid, pl.ds(r0, NRL)])

    return sc_scatter


def kernel(inputs, hidden, edges, node_masks, send_edges, recv_edges,
           edge2node_inds, emb_w, emb_b, edge_w1, edge_b1, edge_w2, edge_b2,
           node_w1, node_b1, node_w2, node_b2, coord_w1, coord_b1, coord_w2,
           vel_w1, vel_b1, vel_w2, vel_b2):
    N = inputs.shape[1]
    E = send_edges.shape[0]
    H = emb_w.shape[1]
    L = edge_w1.shape[0]
    BN = 2000
    BE = 6400
    EH = E // 2
    EW = EH // NW
    CHS = 40
    NCH = EW // CHS

    x = inputs[0]
    xp = jnp.pad(x[:, :3], ((0, 0), (0, DW - 3)))
    xv = jnp.pad(x[:, 3:6], ((0, 0), (0, DW - 3)))
    row = send_edges.astype(jnp.int32)
    col = recv_edges.astype(jnp.int32)
    rows = [row[:EH], row[EH:]]
    cols = [col[:EH], col[EH:]]
    col3s = [c.reshape(NW, EW // 40, 40) for c in cols]
    zp = jnp.zeros((N, PW), F32)

    embw = emb_w.reshape(1, H)
    embb = emb_b.reshape(1, H)
    w1a = [edge_w1[l, :H] for l in range(L)]
    w1b = [edge_w1[l, H:2 * H] for l in range(L)]
    w1c = [edge_w1[l, 2 * H:2 * H + 1] for l in range(L)]
    b1 = [edge_b1[l].reshape(1, H) for l in range(L)]
    w2 = [jnp.pad(edge_w2[l], ((0, 0), (0, PW - H))) for l in range(L)]
    b2 = [jnp.pad(edge_b2[l].reshape(1, H), ((0, 0), (0, PW - H)))
          for l in range(L)]
    cw1 = [jnp.pad(coord_w1[l], ((0, PW - H), (0, 0))) for l in range(L)]
    cb1 = [coord_b1[l].reshape(1, H) for l in range(L)]
    cw2r = [coord_w2[l].reshape(1, H) for l in range(L)]
    vw1 = [vel_w1[l] for l in range(L)]
    vb1 = [vel_b1[l].reshape(1, H) for l in range(L)]
    vw2r = [vel_w2[l].reshape(1, H) for l in range(L)]
    vb2 = [vel_b2[l].reshape(1, 1) for l in range(L)]
    nw1a = [node_w1[l, :H] for l in range(L)]
    nw1b = [node_w1[l, H:] for l in range(L)]
    nb1 = [node_b1[l].reshape(1, H) for l in range(L)]
    nw2 = [node_w2[l] for l in range(L)]
    nb2 = [node_b2[l].reshape(1, H) for l in range(L)]

    CHS2 = 40
    NCH2 = EW // CHS2
    sc_gather = _build_sc_gather(EH, N, EW, CHS, NCH)
    sc_scatter = _build_sc_scatter(EH, N, EW, CHS2, NCH2)

    h, T, U = _tc_init(xp, xv, embw, embb, w1a[0], w1b[0], N, H, BN)
    pos, vel = xp, xv
    for l in range(L):
        Ya = sc_gather(T, U, rows[0], cols[0])
        Yb = sc_gather(T, U, rows[1], cols[1])
        Ea = _tc_edge(Ya, w1c[l], b1[l], w2[l], b2[l], cw1[l], cb1[l],
                      cw2r[l], EH, H, BE)
        p2a = sc_scatter(Ea, col3s[0], zp)
        Eb = _tc_edge(Yb, w1c[l], b1[l], w2[l], b2[l], cw1[l], cb1[l],
                      cw2r[l], EH, H, BE)
        p2b = sc_scatter(Eb, col3s[1], zp)
        nxt = (w1a[l + 1], w1b[l + 1]) if l + 1 < L else None
        outs = _tc_node(h, pos, vel, p2a, p2b, vw1[l], vb1[l], vw2r[l],
                        vb2[l], nw1a[l], nw1b[l], nb1[l], nw2[l], nb2[l],
                        nxt, N, H, BN)
        if nxt is not None:
            h, pos, vel, T, U = outs
        else:
            h, pos, vel = outs

    return jnp.concatenate([pos[:, :3], vel[:, :3]], axis=-1)[None]
